# baseline (speedup 1.0000x reference)
import functools

import jax
import jax.numpy as jnp
from jax.experimental import pallas as pl

B = 16384
INPUT_DIM = 512
EMB_DIM = 128
N_CLUSTERS = 64
BLOCK_ROWS = 1024


def _fused_body(x_ref, pca_ref, cent_ref, out_ref):
    xb = x_ref[...]
    pe = pca_ref[...]
    cen = cent_ref[...]

    x_enc = jax.lax.dot_general(
        xb.astype(jnp.bfloat16), pe.astype(jnp.bfloat16),
        (((1,), (1,)), ((), ())),
        preferred_element_type=jnp.float32)

    cross = jax.lax.dot_general(
        x_enc.astype(jnp.bfloat16), cen.astype(jnp.bfloat16),
        (((1,), (1,)), ((), ())),
        preferred_element_type=jnp.float32)

    x2 = jnp.sum(x_enc * x_enc, axis=1)
    c2 = jnp.sum(cen * cen, axis=1)[None, :]
    m = jnp.min(c2 - 2.0 * cross, axis=1)
    out_ref[...] = jnp.sqrt(jnp.maximum(x2 + m, 0.0))


@functools.partial(jax.jit, static_argnames=("interpret",))
def kernel(x, pca_components, centroids, interpret=False):
    grid = (B // BLOCK_ROWS,)
    return pl.pallas_call(
        _fused_body,
        grid=grid,
        in_specs=[
            pl.BlockSpec((BLOCK_ROWS, INPUT_DIM), lambda i: (i, 0)),
            pl.BlockSpec((EMB_DIM, INPUT_DIM), lambda i: (0, 0)),
            pl.BlockSpec((N_CLUSTERS, EMB_DIM), lambda i: (0, 0)),
        ],
        out_specs=pl.BlockSpec((BLOCK_ROWS,), lambda i: (i,)),
        out_shape=jax.ShapeDtypeStruct((B,), jnp.float32),
        interpret=interpret,
    )(x, pca_components, centroids)


# speedup vs baseline: 10.9962x; 10.9962x over previous
import functools

import jax
import jax.numpy as jnp
from jax.experimental import pallas as pl

B = 16384
INPUT_DIM = 512
EMB_DIM = 128
N_CLUSTERS = 64
BLOCK_ROWS = 1024


def _fused_body(x_ref, pca_ref, cent_ref, out_ref):
    xb = x_ref[...]
    pe = pca_ref[...]
    cen = cent_ref[...]

    x_enc = jax.lax.dot_general(
        xb.astype(jnp.bfloat16), pe.astype(jnp.bfloat16),
        (((1,), (1,)), ((), ())),
        preferred_element_type=jnp.float32)

    cen_p = jnp.concatenate(
        [cen, jnp.zeros((128 - N_CLUSTERS, EMB_DIM), jnp.float32)], axis=0)

    cross = jax.lax.dot_general(
        x_enc.astype(jnp.bfloat16), cen_p.astype(jnp.bfloat16),
        (((1,), (1,)), ((), ())),
        preferred_element_type=jnp.float32)

    x2 = jnp.sum(x_enc * x_enc, axis=1, keepdims=True)
    c2 = jnp.sum(cen_p * cen_p, axis=1)[None, :]
    pad = jax.lax.broadcasted_iota(jnp.int32, (1, 128), 1) >= N_CLUSTERS
    c2 = jnp.where(pad, jnp.float32(3e38), c2)
    m = jnp.min(c2 - 2.0 * cross, axis=1, keepdims=True)
    out_ref[...] = jnp.sqrt(jnp.maximum(x2 + m, 0.0))


@functools.partial(jax.jit, static_argnames=("interpret",))
def kernel(x, pca_components, centroids, interpret=False):
    grid = (B // BLOCK_ROWS,)
    return pl.pallas_call(
        _fused_body,
        grid=grid,
        in_specs=[
            pl.BlockSpec((BLOCK_ROWS, INPUT_DIM), lambda i: (i, 0)),
            pl.BlockSpec((EMB_DIM, INPUT_DIM), lambda i: (0, 0)),
            pl.BlockSpec((N_CLUSTERS, EMB_DIM), lambda i: (0, 0)),
        ],
        out_specs=pl.BlockSpec((BLOCK_ROWS, 1), lambda i: (i, 0)),
        out_shape=jax.ShapeDtypeStruct((B, 1), jnp.float32),
        interpret=interpret,
    )(x, pca_components, centroids).reshape(B)


# speedup vs baseline: 11.3686x; 1.0339x over previous
import functools

import jax
import jax.numpy as jnp
from jax.experimental import pallas as pl
from jax.experimental.pallas import tpu as pltpu

B = 16384
INPUT_DIM = 512
EMB_DIM = 128
N_CLUSTERS = 64
BLOCK_ROWS = 1024


def _fused_body(x_ref, pca_ref, cent_ref, out_ref, xbf_ref, pbf_ref, cbf_ref):
    xbf_ref[...] = x_ref[...].astype(jnp.bfloat16)
    pbf_ref[...] = pca_ref[...].astype(jnp.bfloat16)

    x_enc = jax.lax.dot_general(
        xbf_ref[...], pbf_ref[...],
        (((1,), (1,)), ((), ())),
        preferred_element_type=jnp.float32)

    cen = cent_ref[...]
    cen_p = jnp.concatenate(
        [cen, jnp.zeros((128 - N_CLUSTERS, EMB_DIM), jnp.float32)], axis=0)
    cbf_ref[...] = cen_p.astype(jnp.bfloat16)

    cross = jax.lax.dot_general(
        x_enc.astype(jnp.bfloat16), cbf_ref[...],
        (((1,), (1,)), ((), ())),
        preferred_element_type=jnp.float32)

    x2 = jnp.sum(x_enc * x_enc, axis=1, keepdims=True)
    c2 = jnp.sum(cen_p * cen_p, axis=1)[None, :]
    pad = jax.lax.broadcasted_iota(jnp.int32, (1, 128), 1) >= N_CLUSTERS
    c2 = jnp.where(pad, jnp.float32(3e38), c2)
    m = jnp.min(c2 - 2.0 * cross, axis=1, keepdims=True)
    out_ref[...] = jnp.sqrt(jnp.maximum(x2 + m, 0.0))


@functools.partial(jax.jit, static_argnames=("interpret",))
def kernel(x, pca_components, centroids, interpret=False):
    grid = (B // BLOCK_ROWS,)
    return pl.pallas_call(
        _fused_body,
        grid=grid,
        in_specs=[
            pl.BlockSpec((BLOCK_ROWS, INPUT_DIM), lambda i: (i, 0)),
            pl.BlockSpec((EMB_DIM, INPUT_DIM), lambda i: (0, 0)),
            pl.BlockSpec((N_CLUSTERS, EMB_DIM), lambda i: (0, 0)),
        ],
        out_specs=pl.BlockSpec((BLOCK_ROWS, 1), lambda i: (i, 0)),
        out_shape=jax.ShapeDtypeStruct((B, 1), jnp.float32),
        scratch_shapes=[
            pltpu.VMEM((BLOCK_ROWS, INPUT_DIM), jnp.bfloat16),
            pltpu.VMEM((EMB_DIM, INPUT_DIM), jnp.bfloat16),
            pltpu.VMEM((128, EMB_DIM), jnp.bfloat16),
        ],
        interpret=interpret,
    )(x, pca_components, centroids).reshape(B)


# speedup vs baseline: 13.5127x; 1.1886x over previous
import functools

import jax
import jax.numpy as jnp
from jax.experimental import pallas as pl
from jax.experimental.pallas import tpu as pltpu

B = 16384
INPUT_DIM = 512
EMB_DIM = 128
N_CLUSTERS = 64
BLOCK_ROWS = 2048


def _fused_body(x_ref, pca_ref, cent_ref, out_ref, xbf_ref, pbf_ref, cbf_ref):
    xbf_ref[...] = x_ref[...].astype(jnp.bfloat16)
    pbf_ref[...] = pca_ref[...].astype(jnp.bfloat16)

    x_enc = jax.lax.dot_general(
        xbf_ref[...], pbf_ref[...],
        (((1,), (1,)), ((), ())),
        preferred_element_type=jnp.float32)

    cen = cent_ref[...]
    cen_p = jnp.concatenate(
        [cen, jnp.zeros((128 - N_CLUSTERS, EMB_DIM), jnp.float32)], axis=0)
    cbf_ref[...] = cen_p.astype(jnp.bfloat16)

    cross = jax.lax.dot_general(
        x_enc.astype(jnp.bfloat16), cbf_ref[...],
        (((1,), (1,)), ((), ())),
        preferred_element_type=jnp.float32)

    x2 = jnp.sum(x_enc * x_enc, axis=1, keepdims=True)
    c2 = jnp.sum(cen_p * cen_p, axis=1)[None, :]
    pad = jax.lax.broadcasted_iota(jnp.int32, (1, 128), 1) >= N_CLUSTERS
    c2 = jnp.where(pad, jnp.float32(3e38), c2)
    m = jnp.min(c2 - 2.0 * cross, axis=1, keepdims=True)
    out_ref[...] = jnp.sqrt(jnp.maximum(x2 + m, 0.0))


@functools.partial(jax.jit, static_argnames=("interpret",))
def kernel(x, pca_components, centroids, interpret=False):
    grid = (B // BLOCK_ROWS,)
    return pl.pallas_call(
        _fused_body,
        grid=grid,
        in_specs=[
            pl.BlockSpec((BLOCK_ROWS, INPUT_DIM), lambda i: (i, 0)),
            pl.BlockSpec((EMB_DIM, INPUT_DIM), lambda i: (0, 0)),
            pl.BlockSpec((N_CLUSTERS, EMB_DIM), lambda i: (0, 0)),
        ],
        out_specs=pl.BlockSpec((BLOCK_ROWS, 1), lambda i: (i, 0)),
        out_shape=jax.ShapeDtypeStruct((B, 1), jnp.float32),
        scratch_shapes=[
            pltpu.VMEM((BLOCK_ROWS, INPUT_DIM), jnp.bfloat16),
            pltpu.VMEM((EMB_DIM, INPUT_DIM), jnp.bfloat16),
            pltpu.VMEM((128, EMB_DIM), jnp.bfloat16),
        ],
        interpret=interpret,
    )(x, pca_components, centroids).reshape(B)


# speedup vs baseline: 14.4543x; 1.0697x over previous
import functools

import jax
import jax.numpy as jnp
from jax.experimental import pallas as pl
from jax.experimental.pallas import tpu as pltpu

B = 16384
INPUT_DIM = 512
EMB_DIM = 128
N_CLUSTERS = 64
BLOCK_ROWS = 4096


def _fused_body(x_ref, pca_ref, cent_ref, out_ref, xbf_ref, pbf_ref, cbf_ref):
    xbf_ref[...] = x_ref[...].astype(jnp.bfloat16)
    pbf_ref[...] = pca_ref[...].astype(jnp.bfloat16)

    x_enc = jax.lax.dot_general(
        xbf_ref[...], pbf_ref[...],
        (((1,), (1,)), ((), ())),
        preferred_element_type=jnp.float32)

    cen = cent_ref[...]
    cen_p = jnp.concatenate(
        [cen, jnp.zeros((128 - N_CLUSTERS, EMB_DIM), jnp.float32)], axis=0)
    cbf_ref[...] = cen_p.astype(jnp.bfloat16)

    cross = jax.lax.dot_general(
        x_enc.astype(jnp.bfloat16), cbf_ref[...],
        (((1,), (1,)), ((), ())),
        preferred_element_type=jnp.float32)

    x2 = jnp.sum(x_enc * x_enc, axis=1, keepdims=True)
    c2 = jnp.sum(cen_p * cen_p, axis=1)[None, :]
    pad = jax.lax.broadcasted_iota(jnp.int32, (1, 128), 1) >= N_CLUSTERS
    c2 = jnp.where(pad, jnp.float32(3e38), c2)
    m = jnp.min(c2 - 2.0 * cross, axis=1, keepdims=True)
    out_ref[...] = jnp.sqrt(jnp.maximum(x2 + m, 0.0))


@functools.partial(jax.jit, static_argnames=("interpret",))
def kernel(x, pca_components, centroids, interpret=False):
    grid = (B // BLOCK_ROWS,)
    return pl.pallas_call(
        _fused_body,
        grid=grid,
        in_specs=[
            pl.BlockSpec((BLOCK_ROWS, INPUT_DIM), lambda i: (i, 0)),
            pl.BlockSpec((EMB_DIM, INPUT_DIM), lambda i: (0, 0)),
            pl.BlockSpec((N_CLUSTERS, EMB_DIM), lambda i: (0, 0)),
        ],
        out_specs=pl.BlockSpec((BLOCK_ROWS, 1), lambda i: (i, 0)),
        out_shape=jax.ShapeDtypeStruct((B, 1), jnp.float32),
        scratch_shapes=[
            pltpu.VMEM((BLOCK_ROWS, INPUT_DIM), jnp.bfloat16),
            pltpu.VMEM((EMB_DIM, INPUT_DIM), jnp.bfloat16),
            pltpu.VMEM((128, EMB_DIM), jnp.bfloat16),
        ],
        interpret=interpret,
    )(x, pca_components, centroids).reshape(B)
